# baseline (speedup 1.0000x reference)
import functools

import jax
import jax.numpy as jnp
from jax import lax
from jax.experimental import pallas as pl
from jax.experimental.pallas import tpu as pltpu
from jax.experimental.pallas import tpu_sc as plsc

N = 10000
E = 320000
H = 256
HH = 128
NC, NS = 2, 16
NPAD = 10240
STRIPE = NPAD // NS

F32 = jnp.float32

_mesh = plsc.VectorSubcoreMesh(core_axis_name="c", subcore_axis_name="s")


@functools.partial(
    pl.kernel,
    out_type=jax.ShapeDtypeStruct((NC * NPAD,), F32),
    mesh=_mesh,
    scratch_types=[
        pltpu.VMEM((128,), jnp.int32),
        pltpu.VMEM((128,), F32),
        pltpu.VMEM((16,), jnp.int32),
        pltpu.VMEM((16,), F32),
        pltpu.VMEM((STRIPE,), F32),
        pltpu.VMEM_SHARED((NPAD,), F32),
    ],
)
def _deg_kernel(dst_hbm, out_hbm, dstb, onesb, dstb_t, onesb_t, stage, degsp):
    c = lax.axis_index("c")
    s = lax.axis_index("s")
    for j in range(8):
        onesb[pl.ds(16 * j, 16)] = jnp.full((16,), 1.0, F32)
    onesb_t[pl.ds(0, 16)] = jnp.full((16,), 1.0, F32)
    for j in range(STRIPE // 16):
        stage[pl.ds(16 * j, 16)] = jnp.zeros((16,), F32)
    pltpu.sync_copy(stage, degsp.at[pl.ds(s * STRIPE, STRIPE)])
    plsc.subcore_barrier()

    per_tile = E // (NC * NS)
    base = (s * NC + c) * per_tile
    nfull = per_tile // 128
    tail = per_tile - nfull * 128

    def chunk(k, carry):
        b = base + k * 128
        pltpu.sync_copy(dst_hbm.at[pl.ds(b, 128)], dstb)
        pltpu.sync_copy(onesb, degsp.at[dstb], add=True)
        return carry

    lax.fori_loop(0, nfull, chunk, 0)
    bt = base + nfull * 128
    pltpu.sync_copy(dst_hbm.at[pl.ds(bt, tail)], dstb_t)
    pltpu.sync_copy(onesb_t, degsp.at[dstb_t], add=True)
    plsc.subcore_barrier()

    pltpu.sync_copy(degsp.at[pl.ds(s * STRIPE, STRIPE)], stage)
    pltpu.sync_copy(stage, out_hbm.at[pl.ds(c * NPAD + s * STRIPE, STRIPE)])


ECH_A = 64
PT_A = E // NS
NCH_A = PT_A // ECH_A
ATAIL = PT_A - NCH_A * ECH_A
TR_A = 624
_TR_SPLIT = [(0, 64), (64, 64), (128, 64), (192, 64), (256, 64),
             (320, 64), (384, 64), (448, 64), (512, 64), (576, 48)]


@functools.partial(
    pl.kernel,
    out_type=jax.ShapeDtypeStruct((NC * N, HH), F32),
    mesh=_mesh,
    scratch_types=[
        pltpu.VMEM((PT_A + 96,), jnp.int32),
        pltpu.VMEM((ECH_A,), jnp.int32),
        pltpu.VMEM((ECH_A,), jnp.int32),
        pltpu.VMEM((ECH_A,), jnp.int32),
        pltpu.VMEM((ECH_A, HH), F32),
        pltpu.VMEM((ECH_A, HH), F32),
        pltpu.VMEM((ECH_A, HH), F32),
        pltpu.VMEM((32,), jnp.int32),
        pltpu.VMEM((32, HH), F32),
        pltpu.VMEM_SHARED((N, HH), F32),
        pltpu.SemaphoreType.DMA,
        pltpu.SemaphoreType.DMA,
        pltpu.SemaphoreType.DMA,
        pltpu.SemaphoreType.DMA,
        pltpu.SemaphoreType.DMA,
        pltpu.SemaphoreType.DMA,
        pltpu.SemaphoreType.DMA,
        pltpu.SemaphoreType.DMA,
        pltpu.SemaphoreType.DMA,
    ],
)
def _agg_kernel(hs_hbm, src_hbm, dst_hbm, out_hbm,
                srcall, dstb0, dstb1, dstb2, rows0, rows1, rows2,
                dstb_t, rows_t, aggsp,
                semg0, semg1, semg2, semd0, semd1, semd2,
                sems0, sems1, sems2):
    c = lax.axis_index("c")
    s = lax.axis_index("s")
    rowoff = c * N
    ebase = s * PT_A
    trow = s * TR_A

    def zbody(i, carry):
        for j in range(HH // 16):
            rows0[i, pl.ds(16 * j, 16)] = jnp.zeros((16,), F32)
        return carry

    lax.fori_loop(0, ECH_A, zbody, 0)
    for r, nr in _TR_SPLIT:
        pltpu.sync_copy(rows0.at[pl.ds(0, nr)], aggsp.at[pl.ds(trow + r, nr)])

    @pl.when(s == NS - 1)
    def _():
        pltpu.sync_copy(rows0.at[pl.ds(0, 16)],
                        aggsp.at[pl.ds(NS * TR_A, 16)])

    pltpu.sync_copy(src_hbm.at[pl.ds(ebase, PT_A)], srcall.at[pl.ds(0, PT_A)])

    @pl.when(c == 1)
    def _():
        def abody(i, carry):
            srcall[pl.ds(16 * i, 16)] = srcall[pl.ds(16 * i, 16)] + rowoff
            return carry

        lax.fori_loop(0, PT_A // 16, abody, 0)

    plsc.subcore_barrier()

    bufs = ((rows0, dstb0, semg0, semd0, sems0),
            (rows1, dstb1, semg1, semd1, sems1),
            (rows2, dstb2, semg2, semd2, sems2))

    def issue_in(k, p):
        rows, dstb, sg, sd, _ = bufs[p]
        pltpu.async_copy(hs_hbm.at[srcall.at[pl.ds(k * ECH_A, ECH_A)]], rows, sg)
        pltpu.async_copy(dst_hbm.at[pl.ds(ebase + k * ECH_A, ECH_A)], dstb, sd)

    def wait_in(p):
        rows, dstb, sg, sd, _ = bufs[p]
        pltpu.make_async_copy(hs_hbm.at[pl.ds(0, ECH_A)], rows, sg).wait()
        pltpu.make_async_copy(dst_hbm.at[pl.ds(0, ECH_A)], dstb, sd).wait()

    def wait_scatter(p):
        rows, _, _, _, ss = bufs[p]
        pltpu.make_async_copy(rows, aggsp.at[pl.ds(0, ECH_A)], ss).wait()

    def step(k, p):
        rows, dstb, _, _, ss = bufs[p]
        wait_in(p)
        pltpu.async_copy(rows, aggsp.at[dstb], ss, add=True)
        r = (p + 2) % 3

        @pl.when(k >= 1)
        def _():
            wait_scatter(r)

        @pl.when(k + 2 < NCH_A)
        def _():
            issue_in(k + 2, r)

    issue_in(0, 0)
    issue_in(1, 1)

    def triple(t, carry):
        step(t * 3, 0)
        step(t * 3 + 1, 1)
        step(t * 3 + 2, 2)
        return carry

    lax.fori_loop(0, NCH_A // 3, triple, 0)
    wait_scatter((NCH_A - 1) % 3)

    bt = NCH_A * ECH_A
    cpG = pltpu.async_copy(
        hs_hbm.at[srcall.at[pl.ds(bt, ATAIL)]], rows_t, semg0)
    cpD = pltpu.async_copy(dst_hbm.at[pl.ds(ebase + bt, ATAIL)], dstb_t, semd0)
    cpG.wait()
    cpD.wait()
    pltpu.sync_copy(rows_t, aggsp.at[dstb_t], add=True)
    plsc.subcore_barrier()

    for r, nr in _TR_SPLIT:
        pltpu.sync_copy(aggsp.at[pl.ds(trow + r, nr)], rows0.at[pl.ds(0, nr)])
        pltpu.sync_copy(rows0.at[pl.ds(0, nr)],
                        out_hbm.at[pl.ds(rowoff + trow + r, nr)])

    @pl.when(s == NS - 1)
    def _():
        last = NS * TR_A
        pltpu.sync_copy(aggsp.at[pl.ds(last, 16)], rows_t.at[pl.ds(0, 16)])
        pltpu.sync_copy(rows_t.at[pl.ds(0, 16)],
                        out_hbm.at[pl.ds(rowoff + last, 16)])


ECH_E = 64
PT_E = 9984
PT_LAST = E - 31 * PT_E
SLAB = PT_LAST + 2 * 64
ER = E // 8


@functools.partial(
    pl.kernel,
    out_type=jax.ShapeDtypeStruct((ER, 128), F32),
    mesh=_mesh,
    scratch_types=[
        pltpu.VMEM((SLAB,), jnp.int32),
        pltpu.VMEM((SLAB,), jnp.int32),
        pltpu.VMEM((ECH_E, H), F32),
        pltpu.VMEM((ECH_E, H), F32),
        pltpu.VMEM((ECH_E, H), F32),
        pltpu.VMEM((ECH_E, H), F32),
        pltpu.VMEM((ECH_E, H), F32),
        pltpu.VMEM((ECH_E, H), F32),
        pltpu.VMEM((ECH_E // 8, 128), F32),
        pltpu.VMEM((ECH_E // 8, 128), F32),
        pltpu.VMEM((ECH_E // 8, 128), F32),
        pltpu.VMEM((H,), F32),
        pltpu.SemaphoreType.DMA,
        pltpu.SemaphoreType.DMA,
        pltpu.SemaphoreType.DMA,
        pltpu.SemaphoreType.DMA,
        pltpu.SemaphoreType.DMA,
        pltpu.SemaphoreType.DMA,
        pltpu.SemaphoreType.DMA,
        pltpu.SemaphoreType.DMA,
        pltpu.SemaphoreType.DMA,
    ],
)
def _edge_kernel(a_hbm, b_hbm, src_hbm, dst_hbm, wm2_hbm, out_hbm,
                 srcall, dstall, arows0, brows0, arows1, brows1,
                 arows2, brows2, outc0, outc1, outc2, wmb,
                 semA0, semB0, semA1, semB1, semA2, semB2,
                 semO0, semO1, semO2):
    c = lax.axis_index("c")
    s = lax.axis_index("s")
    wid = s * NC + c
    base = wid * PT_E
    rowbase = wid * (PT_E // 8)
    last_tile = wid == NC * NS - 1
    nch = jnp.where(last_tile, PT_LAST // ECH_E, PT_E // ECH_E)

    pltpu.sync_copy(wm2_hbm, wmb)
    pltpu.sync_copy(src_hbm.at[pl.ds(base, PT_LAST)], srcall.at[pl.ds(0, PT_LAST)])
    pltpu.sync_copy(dst_hbm.at[pl.ds(base, PT_LAST)], dstall.at[pl.ds(0, PT_LAST)])
    wmv = [wmb[pl.ds(16 * j, 16)] for j in range(H // 16)]

    bufs = ((arows0, brows0, outc0, semA0, semB0, semO0),
            (arows1, brows1, outc1, semA1, semB1, semO1),
            (arows2, brows2, outc2, semA2, semB2, semO2))

    def issue(k, p):
        ar, br_, _, sa, sb_, _ = bufs[p]
        pltpu.async_copy(a_hbm.at[srcall.at[pl.ds(k * ECH_E, ECH_E)]], ar, sa)
        pltpu.async_copy(b_hbm.at[dstall.at[pl.ds(k * ECH_E, ECH_E)]], br_, sb_)

    def wait_gather(p):
        ar, br_, _, sa, sb_, _ = bufs[p]
        pltpu.make_async_copy(a_hbm.at[pl.ds(0, ECH_E)], ar, sa).wait()
        pltpu.make_async_copy(a_hbm.at[pl.ds(0, ECH_E)], br_, sb_).wait()

    def wait_out(p):
        _, _, ob, _, _, sO = bufs[p]
        pltpu.make_async_copy(ob, out_hbm.at[pl.ds(0, ECH_E // 8)], sO).wait()

    zero = jnp.zeros((16,), F32)

    def step(k, p):
        ar, br_, ob, _, _, sO = bufs[p]
        wait_gather(p)
        r = (p + 2) % 3

        @pl.when(k + 2 < nch)
        def _():
            issue(k + 2, r)

        @pl.when(k >= 3)
        def _():
            wait_out(p)

        def gbody(g, carry):
            for e in range(8):
                row = g * 8 + e
                acc0 = jnp.zeros((16,), F32)
                acc1 = jnp.zeros((16,), F32)
                for j in range(0, H // 16, 2):
                    va0 = ar[row, pl.ds(16 * j, 16)]
                    vb0 = br_[row, pl.ds(16 * j, 16)]
                    va1 = ar[row, pl.ds(16 * (j + 1), 16)]
                    vb1 = br_[row, pl.ds(16 * (j + 1), 16)]
                    acc0 = acc0 + jnp.maximum(va0 + vb0, zero) * wmv[j]
                    acc1 = acc1 + jnp.maximum(va1 + vb1, zero) * wmv[j + 1]
                ob[g, pl.ds(e * 16, 16)] = acc0 + acc1
            return carry

        lax.fori_loop(0, ECH_E // 8, gbody, 0)
        pltpu.async_copy(ob, out_hbm.at[pl.ds(rowbase + k * (ECH_E // 8),
                                              ECH_E // 8)], sO)

    issue(0, 0)
    issue(1, 1)

    def triple(t, carry):
        step(t * 3, 0)
        step(t * 3 + 1, 1)
        step(t * 3 + 2, 2)
        return carry

    lax.fori_loop(0, PT_E // ECH_E // 3, triple, 0)

    @pl.when(last_tile)
    def _():
        def rtriple(t, carry):
            k = 156 + t * 3
            step(k, 0)
            step(k + 1, 1)
            step(k + 2, 2)
            return carry

        lax.fori_loop(0, 2, rtriple, 0)
        step(162, 0)
        step(163, 1)

    wait_out(0)
    wait_out(1)
    wait_out(2)


RB = 1000
GRID = N // RB


def _dinv_block(dpr):
    deg = dpr[0] + dpr[1] + 1.0
    return lax.rsqrt(jnp.maximum(deg, 1.0))


def _mm_scale_body(xr, wr, br, dpr, outr):
    dinv = _dinv_block(dpr)
    h = jnp.dot(xr[...], wr[...], preferred_element_type=F32) + br[...]
    hs = h * dinv
    outr[0] = hs[:, :HH]
    outr[1] = hs[:, HH:]


def _mm_scale(x, W, b, degp, fin):
    return pl.pallas_call(
        _mm_scale_body,
        grid=(GRID,),
        in_specs=[
            pl.BlockSpec((RB, fin), lambda i: (i, 0)),
            pl.BlockSpec((fin, H), lambda i: (0, 0)),
            pl.BlockSpec((1, H), lambda i: (0, 0)),
            pl.BlockSpec((2, RB, 1), lambda i: (0, i, 0)),
        ],
        out_specs=pl.BlockSpec((2, RB, HH), lambda i: (0, i, 0)),
        out_shape=jax.ShapeDtypeStruct((2, N, HH), F32),
    )(x, W, b, degp)


def _agg_block(aggr, hsr, dpr):
    dinv = _dinv_block(dpr)
    a0 = (aggr[0] + hsr[0]) * dinv
    a1 = (aggr[1] + hsr[1]) * dinv
    return jnp.concatenate([a0, a1], axis=1)


def _stats_body(aggr, hsr, dpr, outr):
    i = pl.program_id(0)
    a = _agg_block(aggr, hsr, dpr)
    blk = jnp.stack([jnp.sum(a, axis=0), jnp.sum(a * a, axis=0)])

    @pl.when(i == 0)
    def _():
        outr[...] = jnp.zeros((2, H), F32)

    outr[...] += blk


def _stats(agg0, hs, degp):
    return pl.pallas_call(
        _stats_body,
        grid=(GRID,),
        in_specs=[
            pl.BlockSpec((2, RB, HH), lambda i: (0, i, 0)),
            pl.BlockSpec((2, RB, HH), lambda i: (0, i, 0)),
            pl.BlockSpec((2, RB, 1), lambda i: (0, i, 0)),
        ],
        out_specs=pl.BlockSpec((2, H), lambda i: (0, 0)),
        out_shape=jax.ShapeDtypeStruct((2, H), F32),
    )(agg0, hs, degp)


def _bn_relu(aggr, hsr, dpr, str_, gr, btr):
    a = _agg_block(aggr, hsr, dpr)
    mean = str_[0] * (1.0 / N)
    var = str_[1] * (1.0 / N) - mean * mean
    xn = gr[...] * (a - mean) * lax.rsqrt(var + 1e-5) + btr[...]
    return jnp.maximum(xn, 0.0)


def _bn_mm_scale_body(aggr, hsr, dpr, str_, gr, btr, wr, br, outr):
    o = _bn_relu(aggr, hsr, dpr, str_, gr, btr)
    h2 = jnp.dot(o, wr[...], preferred_element_type=F32) + br[...]
    hs2 = h2 * _dinv_block(dpr)
    outr[0] = hs2[:, :HH]
    outr[1] = hs2[:, HH:]


def _bn_mm_scale(agg0, hs, degp, stats, g, bt, W, b):
    return pl.pallas_call(
        _bn_mm_scale_body,
        grid=(GRID,),
        in_specs=[
            pl.BlockSpec((2, RB, HH), lambda i: (0, i, 0)),
            pl.BlockSpec((2, RB, HH), lambda i: (0, i, 0)),
            pl.BlockSpec((2, RB, 1), lambda i: (0, i, 0)),
            pl.BlockSpec((2, H), lambda i: (0, 0)),
            pl.BlockSpec((1, H), lambda i: (0, 0)),
            pl.BlockSpec((1, H), lambda i: (0, 0)),
            pl.BlockSpec((H, H), lambda i: (0, 0)),
            pl.BlockSpec((1, H), lambda i: (0, 0)),
        ],
        out_specs=pl.BlockSpec((2, RB, HH), lambda i: (0, i, 0)),
        out_shape=jax.ShapeDtypeStruct((2, N, HH), F32),
    )(agg0, hs, degp, stats, g, bt, W, b)


def _bn_ab_body(aggr, hsr, dpr, str_, gr, btr, war, wbr, bmr, outa, outb):
    h2 = _bn_relu(aggr, hsr, dpr, str_, gr, btr)
    outa[...] = jnp.dot(h2, war[...], preferred_element_type=F32) + bmr[...]
    outb[...] = jnp.dot(h2, wbr[...], preferred_element_type=F32)


def _bn_ab(agg0, hs, degp, stats, g, bt, Wa, Wb, bm):
    return pl.pallas_call(
        _bn_ab_body,
        grid=(GRID,),
        in_specs=[
            pl.BlockSpec((2, RB, HH), lambda i: (0, i, 0)),
            pl.BlockSpec((2, RB, HH), lambda i: (0, i, 0)),
            pl.BlockSpec((2, RB, 1), lambda i: (0, i, 0)),
            pl.BlockSpec((2, H), lambda i: (0, 0)),
            pl.BlockSpec((1, H), lambda i: (0, 0)),
            pl.BlockSpec((1, H), lambda i: (0, 0)),
            pl.BlockSpec((H, H), lambda i: (0, 0)),
            pl.BlockSpec((H, H), lambda i: (0, 0)),
            pl.BlockSpec((1, H), lambda i: (0, 0)),
        ],
        out_specs=[
            pl.BlockSpec((RB, H), lambda i: (i, 0)),
            pl.BlockSpec((RB, H), lambda i: (i, 0)),
        ],
        out_shape=[
            jax.ShapeDtypeStruct((N, H), F32),
            jax.ShapeDtypeStruct((N, H), F32),
        ],
    )(agg0, hs, degp, stats, g, bt, Wa, Wb, bm)


def _finish_body(pr, br, outr):
    x = pr[...]
    rows = lax.broadcasted_iota(jnp.int32, (128, 8), 0) // 16
    cols = lax.broadcasted_iota(jnp.int32, (128, 8), 1)
    m = (rows == cols).astype(F32)
    outr[...] = jnp.dot(x, m, preferred_element_type=F32) + br[...]


FB = 4000


def _finish(partials, bm2):
    return pl.pallas_call(
        _finish_body,
        grid=(E // 8 // FB,),
        in_specs=[
            pl.BlockSpec((FB, 128), lambda i: (i, 0)),
            pl.BlockSpec((1, 1), lambda i: (0, 0)),
        ],
        out_specs=pl.BlockSpec((FB, 8), lambda i: (i, 0)),
        out_shape=jax.ShapeDtypeStruct((E // 8, 8), F32),
    )(partials, bm2)


def kernel(x, edge_index, W1, b1, g1, bt1, W2, b2, g2, bt2, Wm1, bm1, Wm2, bm2):
    src = edge_index[0]
    dst = edge_index[1]
    b1r, g1r, bt1r = b1[None, :], g1[None, :], bt1[None, :]
    b2r, g2r, bt2r = b2[None, :], g2[None, :], bt2[None, :]

    degf = _deg_kernel(dst)
    degp = degf.reshape(NC, NPAD)[:, :N].reshape(NC, N, 1)

    hs1 = _mm_scale(x, W1, b1r, degp, 128)
    agg1 = _agg_kernel(hs1.reshape(NC * N, HH), src, dst)
    agg1 = agg1.reshape(NC, N, HH)
    st1 = _stats(agg1, hs1, degp)
    hs2 = _bn_mm_scale(agg1, hs1, degp, st1, g1r, bt1r, W2, b2r)

    agg2 = _agg_kernel(hs2.reshape(NC * N, HH), src, dst)
    agg2 = agg2.reshape(NC, N, HH)
    st2 = _stats(agg2, hs2, degp)
    A, B = _bn_ab(agg2, hs2, degp, st2, g2r, bt2r, Wm1[:H], Wm1[H:], bm1[None, :])

    wm2 = Wm2[:, 0]
    partial = _edge_kernel(A, B, src, dst, wm2)
    out = _finish(partial, bm2.reshape(1, 1))
    return out.reshape(E, 1)


# speedup vs baseline: 1.0046x; 1.0046x over previous
import functools

import jax
import jax.numpy as jnp
from jax import lax
from jax.experimental import pallas as pl
from jax.experimental.pallas import tpu as pltpu
from jax.experimental.pallas import tpu_sc as plsc

N = 10000
E = 320000
H = 256
HH = 128
NC, NS = 2, 16
NPAD = 10240
STRIPE = NPAD // NS

F32 = jnp.float32

_mesh = plsc.VectorSubcoreMesh(core_axis_name="c", subcore_axis_name="s")


@functools.partial(
    pl.kernel,
    out_type=jax.ShapeDtypeStruct((NC * NPAD,), F32),
    mesh=_mesh,
    scratch_types=[
        pltpu.VMEM((128,), jnp.int32),
        pltpu.VMEM((128,), F32),
        pltpu.VMEM((16,), jnp.int32),
        pltpu.VMEM((16,), F32),
        pltpu.VMEM((STRIPE,), F32),
        pltpu.VMEM_SHARED((NPAD,), F32),
    ],
)
def _deg_kernel(dst_hbm, out_hbm, dstb, onesb, dstb_t, onesb_t, stage, degsp):
    c = lax.axis_index("c")
    s = lax.axis_index("s")
    for j in range(8):
        onesb[pl.ds(16 * j, 16)] = jnp.full((16,), 1.0, F32)
    onesb_t[pl.ds(0, 16)] = jnp.full((16,), 1.0, F32)
    for j in range(STRIPE // 16):
        stage[pl.ds(16 * j, 16)] = jnp.zeros((16,), F32)
    pltpu.sync_copy(stage, degsp.at[pl.ds(s * STRIPE, STRIPE)])
    plsc.subcore_barrier()

    per_tile = E // (NC * NS)
    base = (s * NC + c) * per_tile
    nfull = per_tile // 128
    tail = per_tile - nfull * 128

    def chunk(k, carry):
        b = base + k * 128
        pltpu.sync_copy(dst_hbm.at[pl.ds(b, 128)], dstb)
        pltpu.sync_copy(onesb, degsp.at[dstb], add=True)
        return carry

    lax.fori_loop(0, nfull, chunk, 0)
    bt = base + nfull * 128
    pltpu.sync_copy(dst_hbm.at[pl.ds(bt, tail)], dstb_t)
    pltpu.sync_copy(onesb_t, degsp.at[dstb_t], add=True)
    plsc.subcore_barrier()

    pltpu.sync_copy(degsp.at[pl.ds(s * STRIPE, STRIPE)], stage)
    pltpu.sync_copy(stage, out_hbm.at[pl.ds(c * NPAD + s * STRIPE, STRIPE)])


ECH_A = 64
PT_A = E // NS
NCH_A = PT_A // ECH_A
ATAIL = PT_A - NCH_A * ECH_A
TR_A = 624
_TR_SPLIT = [(0, 64), (64, 64), (128, 64), (192, 64), (256, 64),
             (320, 64), (384, 64), (448, 64), (512, 64), (576, 48)]


@functools.partial(
    pl.kernel,
    out_type=jax.ShapeDtypeStruct((NC * N, HH), F32),
    mesh=_mesh,
    scratch_types=[
        pltpu.VMEM((PT_A + 96,), jnp.int32),
        pltpu.VMEM((ECH_A,), jnp.int32),
        pltpu.VMEM((ECH_A,), jnp.int32),
        pltpu.VMEM((ECH_A,), jnp.int32),
        pltpu.VMEM((ECH_A, HH), F32),
        pltpu.VMEM((ECH_A, HH), F32),
        pltpu.VMEM((ECH_A, HH), F32),
        pltpu.VMEM((32,), jnp.int32),
        pltpu.VMEM((32, HH), F32),
        pltpu.VMEM_SHARED((N, HH), F32),
        pltpu.SemaphoreType.DMA,
        pltpu.SemaphoreType.DMA,
        pltpu.SemaphoreType.DMA,
        pltpu.SemaphoreType.DMA,
        pltpu.SemaphoreType.DMA,
        pltpu.SemaphoreType.DMA,
        pltpu.SemaphoreType.DMA,
        pltpu.SemaphoreType.DMA,
        pltpu.SemaphoreType.DMA,
    ],
)
def _agg_kernel(hs_hbm, src_hbm, dst_hbm, out_hbm,
                srcall, dstb0, dstb1, dstb2, rows0, rows1, rows2,
                dstb_t, rows_t, aggsp,
                semg0, semg1, semg2, semd0, semd1, semd2,
                sems0, sems1, sems2):
    c = lax.axis_index("c")
    s = lax.axis_index("s")
    rowoff = c * N
    ebase = s * PT_A
    trow = s * TR_A

    def zbody(i, carry):
        for j in range(HH // 16):
            rows0[i, pl.ds(16 * j, 16)] = jnp.zeros((16,), F32)
        return carry

    lax.fori_loop(0, ECH_A, zbody, 0)
    for r, nr in _TR_SPLIT:
        pltpu.sync_copy(rows0.at[pl.ds(0, nr)], aggsp.at[pl.ds(trow + r, nr)])

    @pl.when(s == NS - 1)
    def _():
        pltpu.sync_copy(rows0.at[pl.ds(0, 16)],
                        aggsp.at[pl.ds(NS * TR_A, 16)])

    pltpu.sync_copy(src_hbm.at[pl.ds(ebase, PT_A)], srcall.at[pl.ds(0, PT_A)])

    @pl.when(c == 1)
    def _():
        def abody(i, carry):
            srcall[pl.ds(16 * i, 16)] = srcall[pl.ds(16 * i, 16)] + rowoff
            return carry

        lax.fori_loop(0, PT_A // 16, abody, 0)

    plsc.subcore_barrier()

    bufs = ((rows0, dstb0, semg0, semd0, sems0),
            (rows1, dstb1, semg1, semd1, sems1),
            (rows2, dstb2, semg2, semd2, sems2))

    def issue_in(k, p):
        rows, dstb, sg, sd, _ = bufs[p]
        pltpu.async_copy(hs_hbm.at[srcall.at[pl.ds(k * ECH_A, ECH_A)]], rows, sg)
        pltpu.async_copy(dst_hbm.at[pl.ds(ebase + k * ECH_A, ECH_A)], dstb, sd)

    def wait_in(p):
        rows, dstb, sg, sd, _ = bufs[p]
        pltpu.make_async_copy(hs_hbm.at[pl.ds(0, ECH_A)], rows, sg).wait()
        pltpu.make_async_copy(dst_hbm.at[pl.ds(0, ECH_A)], dstb, sd).wait()

    def wait_scatter(p):
        rows, _, _, _, ss = bufs[p]
        pltpu.make_async_copy(rows, aggsp.at[pl.ds(0, ECH_A)], ss).wait()

    def step(k, p):
        rows, dstb, _, _, ss = bufs[p]
        wait_in(p)
        pltpu.async_copy(rows, aggsp.at[dstb], ss, add=True)
        r = (p + 2) % 3

        @pl.when(k >= 1)
        def _():
            wait_scatter(r)

        @pl.when(k + 2 < NCH_A)
        def _():
            issue_in(k + 2, r)

    issue_in(0, 0)
    issue_in(1, 1)

    def triple(t, carry):
        step(t * 3, 0)
        step(t * 3 + 1, 1)
        step(t * 3 + 2, 2)
        return carry

    lax.fori_loop(0, NCH_A // 3, triple, 0)
    wait_scatter((NCH_A - 1) % 3)

    bt = NCH_A * ECH_A
    cpG = pltpu.async_copy(
        hs_hbm.at[srcall.at[pl.ds(bt, ATAIL)]], rows_t, semg0)
    cpD = pltpu.async_copy(dst_hbm.at[pl.ds(ebase + bt, ATAIL)], dstb_t, semd0)
    cpG.wait()
    cpD.wait()
    pltpu.sync_copy(rows_t, aggsp.at[dstb_t], add=True)
    plsc.subcore_barrier()

    for r, nr in _TR_SPLIT:
        pltpu.sync_copy(aggsp.at[pl.ds(trow + r, nr)], rows0.at[pl.ds(0, nr)])
        pltpu.sync_copy(rows0.at[pl.ds(0, nr)],
                        out_hbm.at[pl.ds(rowoff + trow + r, nr)])

    @pl.when(s == NS - 1)
    def _():
        last = NS * TR_A
        pltpu.sync_copy(aggsp.at[pl.ds(last, 16)], rows_t.at[pl.ds(0, 16)])
        pltpu.sync_copy(rows_t.at[pl.ds(0, 16)],
                        out_hbm.at[pl.ds(rowoff + last, 16)])


ECH_E = 64
PT_E = 9984
PT_LAST = E - 31 * PT_E
SLAB = PT_LAST
ER = E // 8


@functools.partial(
    pl.kernel,
    out_type=jax.ShapeDtypeStruct((ER, 128), F32),
    mesh=_mesh,
    scratch_types=[
        pltpu.VMEM((SLAB,), jnp.int32),
        pltpu.VMEM((SLAB,), jnp.int32),
        pltpu.VMEM((ECH_E, H), F32),
        pltpu.VMEM((ECH_E, H), F32),
        pltpu.VMEM((ECH_E, H), F32),
        pltpu.VMEM((ECH_E, H), F32),
        pltpu.VMEM((ECH_E // 8, 128), F32),
        pltpu.VMEM((ECH_E // 8, 128), F32),
        pltpu.VMEM((H,), F32),
        pltpu.SemaphoreType.DMA,
        pltpu.SemaphoreType.DMA,
        pltpu.SemaphoreType.DMA,
        pltpu.SemaphoreType.DMA,
        pltpu.SemaphoreType.DMA,
        pltpu.SemaphoreType.DMA,
    ],
)
def _edge_kernel(a_hbm, b_hbm, src_hbm, dst_hbm, wm2_hbm, out_hbm,
                 srcall, dstall, arows0, brows0, arows1, brows1,
                 outc0, outc1, wmb, semA0, semB0, semA1, semB1, semO0, semO1):
    c = lax.axis_index("c")
    s = lax.axis_index("s")
    wid = s * NC + c
    base = wid * PT_E
    rowbase = wid * (PT_E // 8)
    nch = jnp.where(wid == NC * NS - 1, PT_LAST // ECH_E, PT_E // ECH_E)

    pltpu.sync_copy(wm2_hbm, wmb)
    pltpu.sync_copy(src_hbm.at[pl.ds(base, SLAB)], srcall.at[pl.ds(0, SLAB)])
    pltpu.sync_copy(dst_hbm.at[pl.ds(base, SLAB)], dstall.at[pl.ds(0, SLAB)])
    wmv = [wmb[pl.ds(16 * j, 16)] for j in range(H // 16)]

    bufs = ((arows0, brows0, outc0, semA0, semB0, semO0),
            (arows1, brows1, outc1, semA1, semB1, semO1))

    def issue(k, p):
        ar, br_, _, sa, sb_, _ = bufs[p]
        pltpu.async_copy(a_hbm.at[srcall.at[pl.ds(k * ECH_E, ECH_E)]], ar, sa)
        pltpu.async_copy(b_hbm.at[dstall.at[pl.ds(k * ECH_E, ECH_E)]], br_, sb_)

    def wait_gather(p):
        ar, br_, _, sa, sb_, _ = bufs[p]
        pltpu.make_async_copy(a_hbm.at[pl.ds(0, ECH_E)], ar, sa).wait()
        pltpu.make_async_copy(a_hbm.at[pl.ds(0, ECH_E)], br_, sb_).wait()

    def wait_out(p):
        _, _, ob, _, _, sO = bufs[p]
        pltpu.make_async_copy(ob, out_hbm.at[pl.ds(0, ECH_E // 8)], sO).wait()

    zero = jnp.zeros((16,), F32)

    def handle(k, p):
        ar, br_, ob, _, _, sO = bufs[p]
        wait_gather(p)

        @pl.when(k >= 2)
        def _():
            wait_out(p)

        def gbody(g, carry):
            for e in range(8):
                row = g * 8 + e
                acc0 = jnp.zeros((16,), F32)
                acc1 = jnp.zeros((16,), F32)
                for j in range(0, H // 16, 2):
                    va0 = ar[row, pl.ds(16 * j, 16)]
                    vb0 = br_[row, pl.ds(16 * j, 16)]
                    va1 = ar[row, pl.ds(16 * (j + 1), 16)]
                    vb1 = br_[row, pl.ds(16 * (j + 1), 16)]
                    acc0 = acc0 + jnp.maximum(va0 + vb0, zero) * wmv[j]
                    acc1 = acc1 + jnp.maximum(va1 + vb1, zero) * wmv[j + 1]
                ob[g, pl.ds(e * 16, 16)] = acc0 + acc1
            return carry

        lax.fori_loop(0, ECH_E // 8, gbody, 0)
        pltpu.async_copy(ob, out_hbm.at[pl.ds(rowbase + k * (ECH_E // 8),
                                              ECH_E // 8)], sO)

    issue(0, 0)

    def pair(m, carry):
        k0 = m * 2
        issue(k0 + 1, 1)
        handle(k0, 0)

        @pl.when(k0 + 2 < nch)
        def _():
            issue(k0 + 2, 0)

        handle(k0 + 1, 1)
        return carry

    lax.fori_loop(0, nch // 2, pair, 0)
    wait_out(0)
    wait_out(1)


RB = 1000
GRID = N // RB


def _dinv_block(dpr):
    deg = dpr[0] + dpr[1] + 1.0
    return lax.rsqrt(jnp.maximum(deg, 1.0))


def _mm_scale_body(xr, wr, br, dpr, outr):
    dinv = _dinv_block(dpr)
    h = jnp.dot(xr[...], wr[...], preferred_element_type=F32) + br[...]
    hs = h * dinv
    outr[0] = hs[:, :HH]
    outr[1] = hs[:, HH:]


def _mm_scale(x, W, b, degp, fin):
    return pl.pallas_call(
        _mm_scale_body,
        grid=(GRID,),
        in_specs=[
            pl.BlockSpec((RB, fin), lambda i: (i, 0)),
            pl.BlockSpec((fin, H), lambda i: (0, 0)),
            pl.BlockSpec((1, H), lambda i: (0, 0)),
            pl.BlockSpec((2, RB, 1), lambda i: (0, i, 0)),
        ],
        out_specs=pl.BlockSpec((2, RB, HH), lambda i: (0, i, 0)),
        out_shape=jax.ShapeDtypeStruct((2, N, HH), F32),
    )(x, W, b, degp)


def _agg_block(aggr, hsr, dpr):
    dinv = _dinv_block(dpr)
    a0 = (aggr[0] + hsr[0]) * dinv
    a1 = (aggr[1] + hsr[1]) * dinv
    return jnp.concatenate([a0, a1], axis=1)


def _stats_body(aggr, hsr, dpr, outr):
    i = pl.program_id(0)
    a = _agg_block(aggr, hsr, dpr)
    blk = jnp.stack([jnp.sum(a, axis=0), jnp.sum(a * a, axis=0)])

    @pl.when(i == 0)
    def _():
        outr[...] = jnp.zeros((2, H), F32)

    outr[...] += blk


def _stats(agg0, hs, degp):
    return pl.pallas_call(
        _stats_body,
        grid=(GRID,),
        in_specs=[
            pl.BlockSpec((2, RB, HH), lambda i: (0, i, 0)),
            pl.BlockSpec((2, RB, HH), lambda i: (0, i, 0)),
            pl.BlockSpec((2, RB, 1), lambda i: (0, i, 0)),
        ],
        out_specs=pl.BlockSpec((2, H), lambda i: (0, 0)),
        out_shape=jax.ShapeDtypeStruct((2, H), F32),
    )(agg0, hs, degp)


def _bn_relu(aggr, hsr, dpr, str_, gr, btr):
    a = _agg_block(aggr, hsr, dpr)
    mean = str_[0] * (1.0 / N)
    var = str_[1] * (1.0 / N) - mean * mean
    xn = gr[...] * (a - mean) * lax.rsqrt(var + 1e-5) + btr[...]
    return jnp.maximum(xn, 0.0)


def _bn_mm_scale_body(aggr, hsr, dpr, str_, gr, btr, wr, br, outr):
    o = _bn_relu(aggr, hsr, dpr, str_, gr, btr)
    h2 = jnp.dot(o, wr[...], preferred_element_type=F32) + br[...]
    hs2 = h2 * _dinv_block(dpr)
    outr[0] = hs2[:, :HH]
    outr[1] = hs2[:, HH:]


def _bn_mm_scale(agg0, hs, degp, stats, g, bt, W, b):
    return pl.pallas_call(
        _bn_mm_scale_body,
        grid=(GRID,),
        in_specs=[
            pl.BlockSpec((2, RB, HH), lambda i: (0, i, 0)),
            pl.BlockSpec((2, RB, HH), lambda i: (0, i, 0)),
            pl.BlockSpec((2, RB, 1), lambda i: (0, i, 0)),
            pl.BlockSpec((2, H), lambda i: (0, 0)),
            pl.BlockSpec((1, H), lambda i: (0, 0)),
            pl.BlockSpec((1, H), lambda i: (0, 0)),
            pl.BlockSpec((H, H), lambda i: (0, 0)),
            pl.BlockSpec((1, H), lambda i: (0, 0)),
        ],
        out_specs=pl.BlockSpec((2, RB, HH), lambda i: (0, i, 0)),
        out_shape=jax.ShapeDtypeStruct((2, N, HH), F32),
    )(agg0, hs, degp, stats, g, bt, W, b)


def _bn_ab_body(aggr, hsr, dpr, str_, gr, btr, war, wbr, bmr, outa, outb):
    h2 = _bn_relu(aggr, hsr, dpr, str_, gr, btr)
    outa[...] = jnp.dot(h2, war[...], preferred_element_type=F32) + bmr[...]
    outb[...] = jnp.dot(h2, wbr[...], preferred_element_type=F32)


def _bn_ab(agg0, hs, degp, stats, g, bt, Wa, Wb, bm):
    return pl.pallas_call(
        _bn_ab_body,
        grid=(GRID,),
        in_specs=[
            pl.BlockSpec((2, RB, HH), lambda i: (0, i, 0)),
            pl.BlockSpec((2, RB, HH), lambda i: (0, i, 0)),
            pl.BlockSpec((2, RB, 1), lambda i: (0, i, 0)),
            pl.BlockSpec((2, H), lambda i: (0, 0)),
            pl.BlockSpec((1, H), lambda i: (0, 0)),
            pl.BlockSpec((1, H), lambda i: (0, 0)),
            pl.BlockSpec((H, H), lambda i: (0, 0)),
            pl.BlockSpec((H, H), lambda i: (0, 0)),
            pl.BlockSpec((1, H), lambda i: (0, 0)),
        ],
        out_specs=[
            pl.BlockSpec((RB, H), lambda i: (i, 0)),
            pl.BlockSpec((RB, H), lambda i: (i, 0)),
        ],
        out_shape=[
            jax.ShapeDtypeStruct((N, H), F32),
            jax.ShapeDtypeStruct((N, H), F32),
        ],
    )(agg0, hs, degp, stats, g, bt, Wa, Wb, bm)


def _finish_body(pr, br, outr):
    x = pr[...]
    rows = lax.broadcasted_iota(jnp.int32, (128, 8), 0) // 16
    cols = lax.broadcasted_iota(jnp.int32, (128, 8), 1)
    m = (rows == cols).astype(F32)
    outr[...] = jnp.dot(x, m, preferred_element_type=F32) + br[...]


FB = 4000


def _finish(partials, bm2):
    return pl.pallas_call(
        _finish_body,
        grid=(E // 8 // FB,),
        in_specs=[
            pl.BlockSpec((FB, 128), lambda i: (i, 0)),
            pl.BlockSpec((1, 1), lambda i: (0, 0)),
        ],
        out_specs=pl.BlockSpec((FB, 8), lambda i: (i, 0)),
        out_shape=jax.ShapeDtypeStruct((E // 8, 8), F32),
    )(partials, bm2)


def kernel(x, edge_index, W1, b1, g1, bt1, W2, b2, g2, bt2, Wm1, bm1, Wm2, bm2):
    src = edge_index[0]
    dst = edge_index[1]
    b1r, g1r, bt1r = b1[None, :], g1[None, :], bt1[None, :]
    b2r, g2r, bt2r = b2[None, :], g2[None, :], bt2[None, :]

    degf = _deg_kernel(dst)
    degp = degf.reshape(NC, NPAD)[:, :N].reshape(NC, N, 1)

    hs1 = _mm_scale(x, W1, b1r, degp, 128)
    agg1 = _agg_kernel(hs1.reshape(NC * N, HH), src, dst)
    agg1 = agg1.reshape(NC, N, HH)
    st1 = _stats(agg1, hs1, degp)
    hs2 = _bn_mm_scale(agg1, hs1, degp, st1, g1r, bt1r, W2, b2r)

    agg2 = _agg_kernel(hs2.reshape(NC * N, HH), src, dst)
    agg2 = agg2.reshape(NC, N, HH)
    st2 = _stats(agg2, hs2, degp)
    A, B = _bn_ab(agg2, hs2, degp, st2, g2r, bt2r, Wm1[:H], Wm1[H:], bm1[None, :])

    wm2 = Wm2[:, 0]
    partial = _edge_kernel(A, B, src, dst, wm2)
    out = _finish(partial, bm2.reshape(1, 1))
    return out.reshape(E, 1)
